# baseline (speedup 1.0000x reference)
import functools

import jax
import jax.numpy as jnp
from jax import lax
from jax.experimental import pallas as pl
from jax.experimental.pallas import tpu as pltpu
from jax.experimental.pallas import tpu_sc as plsc

N = 1000000
H = 32
B = 16384

NW = 32
RSPAN = 31232
RBUF = N - (NW - 1) * RSPAN
ROWS_W = RSPAN // 4
NVREG = B // 16
RC = 16
SCH = 976
STAIL = RBUF - RSPAN


def _tc_body(slots_ref, out_ref):
    out_ref[...] = slots_ref[...]


_tc_copy = pl.pallas_call(
    _tc_body,
    grid=(125,),
    in_specs=[pl.BlockSpec((2000, 128), lambda i: (i, 0))],
    out_specs=pl.BlockSpec((2000, 128), lambda i: (i, 0)),
    out_shape=jax.ShapeDtypeStruct((250000, 128), jnp.float32),
)


_sc_mesh = plsc.VectorSubcoreMesh(core_axis_name="c", subcore_axis_name="s")


@functools.partial(
    pl.kernel,
    mesh=_sc_mesh,
    out_type=jax.ShapeDtypeStruct((N,), jnp.float32),
    compiler_params=pltpu.CompilerParams(needs_layout_passes=False),
    scratch_types=[
        pltpu.VMEM((B,), jnp.int32),
        pltpu.VMEM((B,), jnp.float32),
        pltpu.VMEM((RBUF,), jnp.int32),
        pltpu.VMEM((RBUF,), jnp.float32),
        pltpu.VMEM((B,), jnp.int32),
        pltpu.VMEM((RBUF // 4,), jnp.int32),
        pltpu.VMEM((RC, 128), jnp.float32),
        pltpu.VMEM((RC * 4,), jnp.int32),
        pltpu.VMEM((RC * 4,), jnp.int32),
        pltpu.VMEM((16, 128), jnp.float32),
        pltpu.VMEM((SCH,), jnp.float32),
        pltpu.VMEM((SCH,), jnp.float32),
        pltpu.VMEM((16,), jnp.float32),
        pltpu.SemaphoreType.DMA,
        pltpu.SemaphoreType.DMA,
    ],
)
def _sc_scatter(ns2_ref, idx_ref, st_ref, vals2_ref, ws_ref, rw_ref, d_ref,
                slots2_ref, nst_ref, idx_v, ws_v, aux, stv, rlst, raux,
                parent, abuf, avbuf, vrows, wchunk, ochunk, dbuf,
                sem0, sem1):
    wid = lax.axis_index("s") * 2 + lax.axis_index("c")
    lo = wid * RSPAN
    hi = jnp.where(wid == NW - 1, N, lo + RSPAN)
    rowbase = wid * ROWS_W

    pltpu.sync_copy(idx_ref, idx_v)
    pltpu.sync_copy(ws_ref, ws_v)
    pltpu.sync_copy(st_ref.at[pl.ds(lo, RBUF)], stv)
    pltpu.sync_copy(d_ref, dbuf.at[pl.ds(0, 1)])

    iota = lax.broadcasted_iota(jnp.int32, (16,), 0)
    minus1 = jnp.full((16,), -1, jnp.int32)

    def _vgather(v, i):
        return lax.gather(
            v, i[:, None],
            dimension_numbers=lax.GatherDimensionNumbers(
                offset_dims=(), collapsed_slice_dims=(0,),
                start_index_map=(0,)),
            slice_sizes=(1,),
            mode=lax.GatherScatterMode.PROMISE_IN_BOUNDS)

    def _ms_aux(k, c):
        aux[pl.ds(k * 16, 16)] = minus1
        return c
    lax.fori_loop(0, RBUF // 16, _ms_aux, 0)

    def _ms_raux(k, c):
        raux[pl.ds(k * 16, 16)] = minus1
        return c
    lax.fori_loop(0, RBUF // 64, _ms_raux, 0)

    def _pa(b, c):
        j = idx_v[pl.ds(b * 16, 16)]
        m = (j >= lo) & (j < hi)
        jl = jnp.where(m, j - lo, 0)
        iv = b * 16 + iota
        plsc.store_scatter(aux, [jl], iv, mask=m)
        g0 = plsc.load_gather(aux, [jl], mask=m)

        def _fix(_):
            g = plsc.load_gather(aux, [jl], mask=m)
            plsc.store_scatter(aux, [jl], iv, mask=m & (g < iv))
            g2 = plsc.load_gather(aux, [jl], mask=m)
            return jnp.any(m & (g2 < iv))

        lax.while_loop(lambda k: k, _fix, jnp.any(m & (g0 < iv)))
        return c
    lax.fori_loop(0, NVREG, _pa, 0)

    def _pb(b, c):
        j = idx_v[pl.ds(b * 16, 16)]
        m = (j >= lo) & (j < hi)
        jl = jnp.where(m, j - lo, 0)
        iv = b * 16 + iota
        g = plsc.load_gather(aux, [jl], mask=m)
        cur = plsc.load_gather(stv, [jl], mask=m)
        s = ws_v[pl.ds(b * 16, 16)]
        win = m & (g == iv) & (s > cur)
        prl = jnp.where(win, jl >> 2, 0)
        plsc.store_scatter(raux, [prl], iv, mask=win)
        r0 = plsc.load_gather(raux, [prl], mask=win)

        def _fix(_):
            r = plsc.load_gather(raux, [prl], mask=win)
            plsc.store_scatter(raux, [prl], iv, mask=win & (r < iv))
            r2 = plsc.load_gather(raux, [prl], mask=win)
            return jnp.any(win & (r2 < iv))

        lax.while_loop(lambda k: k, _fix, jnp.any(win & (r0 < iv)))
        return c
    lax.fori_loop(0, NVREG, _pb, 0)

    def _pb2(b, n):
        j = idx_v[pl.ds(b * 16, 16)]
        m = (j >= lo) & (j < hi)
        jl = jnp.where(m, j - lo, 0)
        iv = b * 16 + iota
        g = plsc.load_gather(aux, [jl], mask=m)
        cur = plsc.load_gather(stv, [jl], mask=m)
        s = ws_v[pl.ds(b * 16, 16)]
        win = m & (g == iv) & (s > cur)
        prl = jnp.where(win, jl >> 2, 0)
        r = plsc.load_gather(raux, [prl], mask=win)
        rep = win & (r == iv)
        pos = n + plsc.cumsum(rep.astype(jnp.int32)) - 1
        plsc.store_scatter(rlst, [jnp.where(rep, pos, 0)], prl + rowbase,
                           mask=rep)
        return n + plsc.all_reduce_population_count(rep)

    nvec = lax.fori_loop(0, NVREG, _pb2, jnp.zeros((16,), jnp.int32))
    n = jnp.max(nvec)
    dscal = dbuf[pl.ds(0, 16)][0]

    @pl.when(n > 0)
    def _apply():
        nchunks = (n + RC - 1) // RC
        lbase = ((n - 1) // 16) * 16
        lastv = rlst[pl.ds(lbase, 16)]
        lsc = jnp.sum(jnp.where(iota == (n - 1) - lbase, lastv, 0))
        last = jnp.full((16,), lsc, jnp.int32)
        pos = n + iota
        pm = pos < nchunks * RC
        plsc.store_scatter(rlst, [jnp.where(pm, pos, 0)], last, mask=pm)

        def _pc(c, carry):
            prv = rlst[pl.ds(c * RC, 16)]
            cp_par = pltpu.async_copy(slots2_ref.at[prv], parent, sem0)
            for r in range(4):
                q = r * 16 + iota
                pr = _vgather(prv, q >> 2)
                jl = pr * 4 + (q & 3) - lo
                av = plsc.load_gather(aux, [jl])
                cur = plsc.load_gather(stv, [jl])
                avc = jnp.maximum(av, 0)
                sw = plsc.load_gather(ws_v, [avc])
                act = (av >= 0) & (sw > cur)
                abuf[pl.ds(r * 16, 16)] = act.astype(jnp.int32)
                avbuf[pl.ds(r * 16, 16)] = avc
            cp_par.wait()
            for g in range(4):
                a16 = abuf[pl.ds(g * 16, 16)]
                av16 = avbuf[pl.ds(g * 16, 16)]
                cp_v = pltpu.async_copy(vals2_ref.at[av16 >> 2], vrows, sem1)
                cp_v.wait()
                for t in range(16):
                    a_s = a16[t]
                    av_s = av16[t]
                    ordn = g * 16 + t
                    prow = ordn >> 2
                    pcol0 = (ordn & 3) * 32
                    vcol0 = (av_s & 3) * 32

                    @pl.when(a_s > 0)
                    def _patch(t=t, prow=prow, pcol0=pcol0, vcol0=vcol0):
                        parent[prow, pl.ds(pcol0, 16)] = (
                            vrows[t, pl.ds(vcol0, 16)])
                        parent[prow, pl.ds(pcol0 + 16, 16)] = (
                            vrows[t, pl.ds(vcol0 + 16, 16)])
            prv2 = rlst[pl.ds(c * RC, 16)]
            cp_out = pltpu.async_copy(parent, ns2_ref.at[prv2], sem0)
            cp_out.wait()
            return carry

        lax.fori_loop(0, nchunks, _pc, 0)

    def _pd_chunk(base, length):
        cp_w = pltpu.async_copy(rw_ref.at[pl.ds(lo + base, length)], wchunk
                                if length == SCH else wchunk.at[pl.ds(0, length)],
                                sem1)
        cp_w.wait()
        for v in range(length // 16):
            x = base + v * 16
            av = aux[pl.ds(x, 16)]
            cur = stv[pl.ds(x, 16)]
            avc = jnp.maximum(av, 0)
            sw = plsc.load_gather(ws_v, [avc])
            act = av >= 0
            val = jnp.where(act & (sw > cur), sw, cur)
            w = wchunk[pl.ds(v * 16, 16)]
            ochunk[pl.ds(v * 16, 16)] = jnp.clip(val + dscal * w, 0.0, 1.0)
        src = ochunk if length == SCH else ochunk.at[pl.ds(0, length)]
        cp_o = pltpu.async_copy(src, nst_ref.at[pl.ds(lo + base, length)], sem1)
        cp_o.wait()

    def _pd(c, carry):
        _pd_chunk(c * SCH, SCH)
        return carry
    lax.fori_loop(0, RSPAN // SCH, _pd, 0)

    @pl.when(wid == NW - 1)
    def _tail():
        _pd_chunk(RSPAN, STAIL)


def kernel(slots, strength, vals, write_strengths, retrieval_weights, delta, idx):
    idx = idx.astype(jnp.int32)
    slots2 = slots.reshape(250000, 128)
    vals2 = vals.reshape(4096, 128)
    slots_copy = _tc_copy(slots2)
    ref_s = jax.new_ref(slots_copy)
    new_strength = _sc_scatter(ref_s, idx, strength, vals2, write_strengths,
                               retrieval_weights, delta, slots2)
    return jax.freeze(ref_s).reshape(N, H), new_strength
